# Initial kernel scaffold; baseline (speedup 1.0000x reference)
import jax
import jax.numpy as jnp
from jax.experimental import pallas as pl


def kernel(x, edge_index, cluster, W1_0, b1_0, g_0, be_0, W2_0, b2_0, W1_1, b1_1, g_1, be_1, W2_1, b2_1, W1_2, b1_2, g_2, be_2, W2_2, b2_2):
    raise NotImplementedError("write your pallas kernel here")


# speedup vs baseline: 1.3661x; 1.3661x over previous
import functools

import jax
import jax.numpy as jnp
from jax import lax
from jax.experimental import pallas as pl
from jax.experimental.pallas import tpu as pltpu
from jax.experimental.pallas import tpu_sc as plsc

N = 10000
E = 320000
D = 128
H = 64
C = 1000

NC = 2
NS = 16
NW = NC * NS

NB_E = 64
RNG_E = 160
MAG_E = 26215
NPAD = NB_E * RNG_E
NB_C = 32
RNG_C = 32
MAG_C = 131072
CPAD = NB_C * RNG_C

KB = 1000
KB_C = 640
SLACK = 480
EPAD = E + SLACK
NPAD2 = 10240
SLACK_C = 320

NEG = -3.0e38


def _mlp_body(nin, a_ref, b_ref, w1a_ref, w1b_ref, b1_ref, g_ref, be_ref,
              w2_ref, b2_ref, o_ref):
    h = jnp.dot(a_ref[...], w1a_ref[...], preferred_element_type=jnp.float32)
    if nin == 2:
        h = h + jnp.dot(b_ref[...], w1b_ref[...],
                        preferred_element_type=jnp.float32)
    h = h + b1_ref[...]
    mu = jnp.mean(h, axis=-1, keepdims=True)
    var = jnp.mean((h - mu) ** 2, axis=-1, keepdims=True)
    h = (h - mu) * lax.rsqrt(var + 1e-5) * g_ref[...] + be_ref[...]
    h = jnp.maximum(h, 0.0)
    o_ref[...] = (jnp.dot(h, w2_ref[...], preferred_element_type=jnp.float32)
                  + b2_ref[...])


def _mlp(a, b, W1, b1, g, be, W2, b2):
    din_a = a.shape[1]
    dout = W2.shape[1]
    nin = 1 if b is None else 2
    w1a = W1[:din_a]
    w1b = W1[din_a:] if nin == 2 else W1[:1]
    bdummy = a[:, :1] if b is None else b
    BN = 2000
    grid = (N // BN,)

    full = lambda s: pl.BlockSpec(s, lambda i: (0, 0))
    return pl.pallas_call(
        functools.partial(_mlp_body, nin),
        grid=grid,
        in_specs=[
            pl.BlockSpec((BN, din_a), lambda i: (i, 0)),
            pl.BlockSpec((BN, bdummy.shape[1]), lambda i: (i, 0)),
            full(w1a.shape),
            full(w1b.shape),
            full((1, H)),
            full((1, H)),
            full((1, H)),
            full(W2.shape),
            full((1, dout)),
        ],
        out_specs=pl.BlockSpec((BN, dout), lambda i: (i, 0)),
        out_shape=jax.ShapeDtypeStruct((N, dout), jnp.float32),
    )(a, bdummy, w1a, w1b, b1.reshape(1, H), g.reshape(1, H),
      be.reshape(1, H), W2, b2.reshape(1, dout))


def _rank1_body(kb, nb, mag, d_ref, l_ref, rank_ref, wcnt_ref):
    d = d_ref[...]
    b = lax.shift_right_logical(d * mag, 22)
    cols = lax.broadcasted_iota(jnp.int32, (kb, nb), 1)
    oh = jnp.where(b == cols, 1.0, 0.0)
    incl = jnp.dot(l_ref[...], oh, preferred_element_type=jnp.float32)
    rank_ref[...] = jnp.sum(oh * (incl - 1.0), axis=1, keepdims=True)
    wcnt_ref[...] = incl[kb - 1:kb, :].reshape(1, 1, nb)


def _rank2_body(w_ref, excl_ref, base_ref, tot_ref):
    w = w_ref[...]
    nwin, nb = w.shape
    v = w
    k = 1
    while k < nwin:
        v = v + jnp.concatenate(
            [jnp.zeros((k, nb), jnp.float32), v[:-k]], axis=0)
        k *= 2
    excl_ref[...] = v - w
    tot = v[nwin - 1:nwin, :]
    u = tot
    k = 1
    while k < nb:
        u = u + jnp.concatenate(
            [jnp.zeros((1, k), jnp.float32), u[:, :-k]], axis=1)
        k *= 2
    base_ref[...] = u - tot
    tot_ref[...] = tot


def _rank3_body(kb, nb, mag, d_ref, rank_ref, excl_ref, base_ref, pos_ref):
    d = d_ref[...]
    b = lax.shift_right_logical(d * mag, 22)
    cols = lax.broadcasted_iota(jnp.int32, (kb, nb), 1)
    oh = jnp.where(b == cols, 1.0, 0.0)
    off = excl_ref[...].reshape(1, nb) + base_ref[...]
    posf = rank_ref[...] + jnp.sum(oh * off, axis=1, keepdims=True)
    pos_ref[...] = posf.astype(jnp.int32)


def _ranking(dstcol, nb, mag, kb):
    M = dstcol.shape[0]
    W = M // kb
    tril = jnp.tril(jnp.ones((kb, kb), jnp.float32))
    rank, wcnt = pl.pallas_call(
        functools.partial(_rank1_body, kb, nb, mag),
        grid=(W,),
        in_specs=[
            pl.BlockSpec((kb, 1), lambda i: (i, 0)),
            pl.BlockSpec((kb, kb), lambda i: (0, 0)),
        ],
        out_specs=[
            pl.BlockSpec((kb, 1), lambda i: (i, 0)),
            pl.BlockSpec((1, 1, nb), lambda i: (i, 0, 0)),
        ],
        out_shape=[
            jax.ShapeDtypeStruct((M, 1), jnp.float32),
            jax.ShapeDtypeStruct((W, 1, nb), jnp.float32),
        ],
    )(dstcol, tril)
    wcnt = wcnt.reshape(W, nb)
    excl, base, tot = pl.pallas_call(
        _rank2_body,
        out_shape=[
            jax.ShapeDtypeStruct((W, nb), jnp.float32),
            jax.ShapeDtypeStruct((1, nb), jnp.float32),
            jax.ShapeDtypeStruct((1, nb), jnp.float32),
        ],
    )(wcnt)
    bmeta = jnp.broadcast_to(
        base.astype(jnp.int32).reshape(nb, 1), (nb, 16))
    cmeta = jnp.broadcast_to(
        tot.astype(jnp.int32).reshape(nb, 1), (nb, 16))
    pos = pl.pallas_call(
        functools.partial(_rank3_body, kb, nb, mag),
        grid=(W,),
        in_specs=[
            pl.BlockSpec((kb, 1), lambda i: (i, 0)),
            pl.BlockSpec((kb, 1), lambda i: (i, 0)),
            pl.BlockSpec((1, 1, nb), lambda i: (i, 0, 0)),
            pl.BlockSpec((1, nb), lambda i: (0, 0)),
        ],
        out_specs=pl.BlockSpec((kb, 1), lambda i: (i, 0)),
        out_shape=jax.ShapeDtypeStruct((M, 1), jnp.int32),
    )(dstcol, rank, excl.reshape(W, 1, nb), base)
    return pos, bmeta, cmeta


def _permute_body(m, mpad, src_hbm, dst_hbm, pos_hbm, ssrc_hbm, sdst_hbm,
                  vbuf, dbuf, pbuf, sem):
    wid = lax.axis_index("c") * NS + lax.axis_index("s")
    pw = vbuf.shape[0]
    per = m // NW

    def win_body(w, _):
        off = wid * per + w * pw
        pltpu.sync_copy(src_hbm.at[pl.ds(off, pw)], vbuf)
        pltpu.sync_copy(dst_hbm.at[pl.ds(off, pw)], dbuf)
        pltpu.sync_copy(pos_hbm.at[pl.ds(off, pw)], pbuf)
        cp = pltpu.make_async_copy(vbuf, ssrc_hbm.at[pbuf], sem)
        cp.start()
        cp.wait()
        cp = pltpu.make_async_copy(dbuf, sdst_hbm.at[pbuf], sem)
        cp.start()
        cp.wait()
        return 0

    lax.fori_loop(0, per // pw, win_body, 0)

    nfill = mpad - m

    @pl.when(wid == 0)
    def _():
        def zb(i, _):
            vbuf[pl.ds(i * 16, 16)] = jnp.zeros((16,), jnp.int32)
            dbuf[pl.ds(i * 16, 16)] = jnp.full((16,), -1, jnp.int32)
            return 0
        lax.fori_loop(0, pw // 16, zb, 0)

        def fb(f, _):
            pltpu.sync_copy(vbuf, ssrc_hbm.at[pl.ds(m + f * pw, pw)])
            pltpu.sync_copy(dbuf, sdst_hbm.at[pl.ds(m + f * pw, pw)])
            return 0
        lax.fori_loop(0, nfill // pw, fb, 0)


def _permute(src, dst, pos, mpad, pw):
    m = src.shape[0]
    mesh = plsc.VectorSubcoreMesh(core_axis_name="c", subcore_axis_name="s")
    f = pl.kernel(
        functools.partial(_permute_body, m, mpad),
        out_type=(jax.ShapeDtypeStruct((mpad,), jnp.int32),
                  jax.ShapeDtypeStruct((mpad,), jnp.int32)),
        mesh=mesh,
        scratch_types=[
            pltpu.VMEM((pw,), jnp.int32),
            pltpu.VMEM((pw,), jnp.int32),
            pltpu.VMEM((pw,), jnp.int32),
            pltpu.SemaphoreType.DMA,
        ],
    )
    return f(src, dst, pos)


def _segmax_body(din, rng, npass, pw, nsrc,
                 ssrc_hbm, sdst_hbm, bmeta_hbm, cmeta_hbm, x_hbm, out_hbm,
                 ibuf, dbuf, mbuf, rows, aggbuf, sem):
    wid = lax.axis_index("c") * NS + lax.axis_index("s")
    nck = din // 16
    lane = jnp.arange(16, dtype=jnp.int32)

    for p in range(npass):
        bkt = wid * npass + p
        lo = bkt * rng
        pltpu.sync_copy(bmeta_hbm.at[bkt], mbuf)
        base = mbuf[pl.ds(0, 16)][0]
        pltpu.sync_copy(cmeta_hbm.at[bkt], mbuf)
        cnt = mbuf[pl.ds(0, 16)][0]
        start = (base // 8) * 8
        end = base + cnt

        def init_body(r, _):
            for c2 in range(nck):
                aggbuf[r, pl.ds(c2 * 16, 16)] = jnp.full((16,), NEG,
                                                         jnp.float32)
            return 0
        lax.fori_loop(0, rng + 1, init_body, 0)

        def win_body(w, _):
            off = start + w * pw
            pltpu.sync_copy(ssrc_hbm.at[pl.ds(off, pw)], ibuf)
            pltpu.sync_copy(sdst_hbm.at[pl.ds(off, pw)], dbuf)

            def fix_body(k, _):
                sv = ibuf[pl.ds(k * 16, 16)]
                ibuf[pl.ds(k * 16, 16)] = jnp.minimum(
                    jnp.maximum(sv, 0), nsrc - 1)
                pg = off + k * 16 + lane
                vi = jnp.minimum(jnp.maximum(
                    jnp.minimum(pg - base + 1, end - pg), 0), 1)
                t = dbuf[pl.ds(k * 16, 16)] - lo
                t2 = vi * t + (1 - vi) * rng
                dbuf[pl.ds(k * 16, 16)] = jnp.minimum(
                    jnp.maximum(t2, 0), rng)
                return 0
            lax.fori_loop(0, pw // 16, fix_body, 0)

            cp = pltpu.make_async_copy(x_hbm.at[ibuf], rows, sem)
            cp.start()
            cp.wait()

            def grp_body(j, _):
                dvec = dbuf[pl.ds(j * 16, 16)]
                for q in range(16):
                    t = dvec[q]
                    for c2 in range(nck):
                        a = aggbuf[t, pl.ds(c2 * 16, 16)]
                        r = rows[j * 16 + q, pl.ds(c2 * 16, 16)]
                        aggbuf[t, pl.ds(c2 * 16, 16)] = jnp.maximum(a, r)
                return 0
            lax.fori_loop(0, pw // 16, grp_body, 0)
            return 0

        nwin = (end - start + pw - 1) // pw
        lax.fori_loop(0, nwin, win_body, 0)

        def fin_body(r, _):
            for c2 in range(nck):
                v = aggbuf[r, pl.ds(c2 * 16, 16)]
                aggbuf[r, pl.ds(c2 * 16, 16)] = jnp.where(v > NEG, v, 0.0)
            return 0
        lax.fori_loop(0, rng, fin_body, 0)
        pltpu.sync_copy(aggbuf.at[pl.ds(0, rng)], out_hbm.at[pl.ds(lo, rng)])


def _segmax(ssrc, sdst, bmeta, cmeta, x, nseg, rng, npass, pw):
    din = x.shape[1]
    mesh = plsc.VectorSubcoreMesh(core_axis_name="c", subcore_axis_name="s")
    f = pl.kernel(
        functools.partial(_segmax_body, din, rng, npass, pw, x.shape[0]),
        out_type=jax.ShapeDtypeStruct((nseg, din), jnp.float32),
        mesh=mesh,
        scratch_types=[
            pltpu.VMEM((pw,), jnp.int32),
            pltpu.VMEM((pw,), jnp.int32),
            pltpu.VMEM((16,), jnp.int32),
            pltpu.VMEM((pw, din), jnp.float32),
            pltpu.VMEM((rng + 1, din), jnp.float32),
            pltpu.SemaphoreType.DMA,
        ],
    )
    return f(ssrc, sdst, bmeta, cmeta, x)


def _norm_body(p_ref, o_ref):
    p = p_ref[...]
    ss = jnp.sum(p * p, axis=0, keepdims=True)
    o_ref[...] = p[:C, :] / jnp.sqrt(ss)


def _normalize(pooled):
    return pl.pallas_call(
        _norm_body,
        out_shape=jax.ShapeDtypeStruct((C, pooled.shape[1]), jnp.float32),
    )(pooled)


def kernel(x, edge_index, cluster, W1_0, b1_0, g_0, be_0, W2_0, b2_0,
           W1_1, b1_1, g_1, be_1, W2_1, b2_1,
           W1_2, b1_2, g_2, be_2, W2_2, b2_2):
    src = edge_index[0]
    dst = edge_index[1]

    pos_e, bmeta_e, cmeta_e = _ranking(dst.reshape(E, 1), NB_E, MAG_E, KB)
    ssrc, sdst = _permute(src, dst, pos_e.reshape(E), EPAD, 80)

    nid = jnp.arange(NPAD2, dtype=jnp.int32)
    clpad = jnp.concatenate(
        [cluster, jnp.full((NPAD2 - N,), CPAD, jnp.int32)])
    pos_c, bmeta_c, cmeta_c = _ranking(clpad.reshape(NPAD2, 1), NB_C + 1,
                                       MAG_C, KB_C)
    snid, scl = _permute(nid, clpad, pos_c.reshape(NPAD2),
                         NPAD2 + SLACK_C, 80)

    seg = lambda xx, pw: _segmax(ssrc, sdst, bmeta_e, cmeta_e, xx,
                                 NPAD, RNG_E, 2, pw)[:N]
    pool = lambda xx: _segmax(snid, scl, bmeta_c, cmeta_c, xx,
                              CPAD, RNG_C, 1, 64)

    a0 = _mlp(x, None, W1_0, b1_0, g_0, be_0, W2_0, b2_0)
    g0 = seg(a0, 64)
    a1 = _mlp(a0, g0, W1_1, b1_1, g_1, be_1, W2_1, b2_1)
    g1 = seg(a1, 64)
    a2 = _mlp(a1, g1, W1_2, b1_2, g_2, be_2, W2_2, b2_2)
    g2 = seg(a2, 64)

    pa = pool(a2)
    pb = pool(g2)
    na = _normalize(pa)
    nb = _normalize(pb)
    return jnp.concatenate([na, nb], axis=1)
